# baseline (speedup 1.0000x reference)
import jax
import jax.numpy as jnp
from jax.experimental import pallas as pl
from jax.experimental.pallas import tpu as pltpu
from jax.experimental.pallas import tpu_sc as plsc

WINDOW = 128


def kernel(input_tokens, table):
    B, S = input_tokens.shape
    V, D = table.shape
    num_indices = B * S
    DP = 2 * D

    flat_idx = input_tokens.reshape(num_indices)
    pair_idx = (flat_idx >> 1).reshape(1, num_indices)
    tab2 = table.reshape(V // 2, DP)

    mesh = plsc.VectorSubcoreMesh(core_axis_name="core", subcore_axis_name="subcore")

    @pl.kernel(
        out_type=jax.ShapeDtypeStruct((num_indices, DP), table.dtype),
        mesh=mesh,
    )
    def gather_kernel(tab_hbm, idx_hbm, out_hbm):
        def body(i_vmem, o_vmem):
            pltpu.sync_copy(tab_hbm.at[i_vmem.at[0]], o_vmem)

        pltpu.emit_pipeline(
            body,
            grid=(num_indices // WINDOW,),
            in_specs=[pl.BlockSpec((1, WINDOW), index_map=lambda i: (0, i))],
            out_specs=[pl.BlockSpec((WINDOW, DP), index_map=lambda i: (i, 0))],
            core_axis_name=("core", "subcore"),
            dimension_semantics=(pltpu.PARALLEL,),
        )(idx_hbm, out_hbm)

    pairs = gather_kernel(tab2, pair_idx)
    odd = (flat_idx & 1).astype(jnp.bool_)[:, None]
    out = jnp.where(odd, pairs[:, D:], pairs[:, :D])
    return out.reshape(B, S, D)


# speedup vs baseline: 1.3904x; 1.3904x over previous
import dataclasses

import jax
import jax.numpy as jnp
from jax import lax
from jax.experimental import pallas as pl
from jax.experimental.pallas import tpu as pltpu
from jax.experimental.pallas import tpu_sc as plsc

TCOLS = 4096
WINDOW = 128


def _repack_table(tab_t):
    D, V = tab_t.shape

    def body(in_ref, out_ref):
        out_ref[:, :D] = in_ref[...].T

    return pl.pallas_call(
        body,
        grid=(pl.cdiv(V, TCOLS),),
        in_specs=[pl.BlockSpec((D, TCOLS), lambda i: (0, i))],
        out_specs=pl.BlockSpec((TCOLS, 128), lambda i: (i, 0)),
        out_shape=jax.ShapeDtypeStruct((V, 128), tab_t.dtype),
    )(tab_t)


def kernel(input_tokens, table):
    B, S = input_tokens.shape
    V, D = table.shape
    num_windows = (B * S) // WINDOW
    n_workers = 32
    per_w = num_windows // n_workers
    bw = B // WINDOW

    tab_p = _repack_table(table.T)
    idx_t = input_tokens.T

    mesh = plsc.VectorSubcoreMesh(core_axis_name="core", subcore_axis_name="subcore")

    cp = pltpu.CompilerParams()
    if "needs_layout_passes" in pltpu.CompilerParams.__dataclass_fields__:
        cp = dataclasses.replace(cp, needs_layout_passes=False)

    @pl.kernel(
        out_type=jax.ShapeDtypeStruct((S, D, B), table.dtype),
        mesh=mesh,
        scratch_types=[
            pltpu.VMEM((2, WINDOW), jnp.int32),
            pltpu.VMEM((2, WINDOW, 128), table.dtype),
            pltpu.VMEM((2, D, WINDOW), table.dtype),
            pltpu.SemaphoreType.DMA,
            pltpu.SemaphoreType.DMA,
        ],
        compiler_params=cp,
    )
    def gather_kernel(tab_hbm, idx_hbm, out_hbm, idx_v, g_v, t_v, gsem, osem):
        wid = lax.axis_index("subcore") * 2 + lax.axis_index("core")
        base = wid * per_w
        iota16 = lax.iota(jnp.int32, 16)

        def out_slice(w):
            return out_hbm.at[w // bw, :, pl.ds((w % bw) * WINDOW, WINDOW)]

        def load_idx_and_fire(n, slot):
            w = base + n
            pltpu.sync_copy(
                idx_hbm.at[w // bw, pl.ds((w % bw) * WINDOW, WINDOW)],
                idx_v.at[slot],
            )
            pltpu.async_copy(tab_hbm.at[idx_v.at[slot]], g_v.at[slot], gsem)

        def transpose(slot):
            @pl.loop(0, WINDOW, step=16)
            def _(q):
                rows = q + iota16

                @pl.loop(0, D, step=8)
                def _(d0):
                    for dd in range(8):
                        cols = jnp.full((16,), d0 + dd, jnp.int32)
                        t_v[slot, d0 + dd, pl.ds(q, 16)] = plsc.load_gather(
                            g_v.at[slot], [rows, cols]
                        )

        def step(n, slot):
            @pl.when(n + 1 < per_w)
            def _():
                load_idx_and_fire(n + 1, 1 - slot)

            pltpu.make_async_copy(
                tab_hbm.at[idx_v.at[slot]], g_v.at[slot], gsem
            ).wait()

            @pl.when(n >= 2)
            def _():
                pltpu.make_async_copy(t_v.at[slot], out_slice(base + n - 2), osem).wait()

            transpose(slot)
            pltpu.async_copy(t_v.at[slot], out_slice(base + n), osem)

        load_idx_and_fire(0, 0)

        @pl.loop(0, per_w, step=2)
        def _(n):
            step(n, 0)
            step(n + 1, 1)

        pltpu.make_async_copy(t_v.at[0], out_slice(base + per_w - 2), osem).wait()
        pltpu.make_async_copy(t_v.at[1], out_slice(base + per_w - 1), osem).wait()

    out_sdb = gather_kernel(tab_p, idx_t)
    return jnp.transpose(out_sdb, (2, 0, 1))


# speedup vs baseline: 1.4830x; 1.0666x over previous
import jax
import jax.numpy as jnp
from jax.experimental import pallas as pl
from jax.experimental.pallas import tpu as pltpu
from jax.experimental.pallas import tpu_sc as plsc

TCOLS = 4096
WINDOW = 128
WB = 128


def _repack_table(tab_t):
    D, V = tab_t.shape

    def body(in_ref, out_ref):
        out_ref[:, :D] = in_ref[...].T

    return pl.pallas_call(
        body,
        grid=(pl.cdiv(V, TCOLS),),
        in_specs=[pl.BlockSpec((D, TCOLS), lambda i: (0, i))],
        out_specs=pl.BlockSpec((TCOLS, 128), lambda i: (i, 0)),
        out_shape=jax.ShapeDtypeStruct((V, 128), tab_t.dtype),
    )(tab_t)


def _tail_transpose(rows2, B, S, D):
    def body(in_ref, out_ref):
        x = in_ref[...]
        for s in range(S):
            out_ref[s] = x[:, s * 128 : s * 128 + D].T

    return pl.pallas_call(
        body,
        grid=(B // WB,),
        in_specs=[pl.BlockSpec((WB, S * 128), lambda b: (b, 0))],
        out_specs=pl.BlockSpec((S, D, WB), lambda b: (0, 0, b)),
        out_shape=jax.ShapeDtypeStruct((S, D, B), rows2.dtype),
    )(rows2)


def kernel(input_tokens, table):
    B, S = input_tokens.shape
    V, D = table.shape
    num_indices = B * S

    tab_p = _repack_table(table.T)
    flat_idx = input_tokens.reshape(1, num_indices)

    mesh = plsc.VectorSubcoreMesh(core_axis_name="core", subcore_axis_name="subcore")

    @pl.kernel(
        out_type=jax.ShapeDtypeStruct((num_indices, 128), table.dtype),
        mesh=mesh,
    )
    def gather_kernel(tab_hbm, idx_hbm, out_hbm):
        def body(i_vmem, o_vmem):
            pltpu.sync_copy(tab_hbm.at[i_vmem.at[0]], o_vmem)

        pltpu.emit_pipeline(
            body,
            grid=(num_indices // WINDOW,),
            in_specs=[pl.BlockSpec((1, WINDOW), index_map=lambda i: (0, i))],
            out_specs=[pl.BlockSpec((WINDOW, 128), index_map=lambda i: (i, 0))],
            core_axis_name=("core", "subcore"),
            dimension_semantics=(pltpu.PARALLEL,),
        )(idx_hbm, out_hbm)

    rows = gather_kernel(tab_p, flat_idx)
    out_sdb = _tail_transpose(rows.reshape(B, S * 128), B, S, D)
    return jnp.transpose(out_sdb, (2, 0, 1))


# speedup vs baseline: 2.1017x; 1.4172x over previous
import jax
import jax.numpy as jnp
from jax.experimental import pallas as pl
from jax.experimental.pallas import tpu as pltpu
from jax.experimental.pallas import tpu_sc as plsc

TCOLS = 8192
WINDOW = 128
WB = 2048


def _repack_table(tab_t):
    D, V = tab_t.shape

    def body(in_ref, out_ref):
        out_ref[:, :D] = in_ref[...].T

    return pl.pallas_call(
        body,
        grid=(pl.cdiv(V, TCOLS),),
        in_specs=[pl.BlockSpec((D, TCOLS), lambda i: (0, i))],
        out_specs=pl.BlockSpec((TCOLS, 128), lambda i: (i, 0)),
        out_shape=jax.ShapeDtypeStruct((V, 128), tab_t.dtype),
    )(tab_t)


def _tail_transpose(rows3, B, S, D):
    def body(in_ref, out_ref):
        x = in_ref[0]
        out_ref[0] = x[:, :D].T

    return pl.pallas_call(
        body,
        grid=(S, B // WB),
        in_specs=[pl.BlockSpec((1, WB, 128), lambda s, b: (s, b, 0))],
        out_specs=pl.BlockSpec((1, D, WB), lambda s, b: (s, 0, b)),
        out_shape=jax.ShapeDtypeStruct((S, D, B), rows3.dtype),
    )(rows3)


def kernel(input_tokens, table):
    B, S = input_tokens.shape
    V, D = table.shape

    tab_p = _repack_table(table.T)
    idx_t = input_tokens.T

    mesh = plsc.VectorSubcoreMesh(core_axis_name="core", subcore_axis_name="subcore")

    @pl.kernel(
        out_type=jax.ShapeDtypeStruct((S, B, 128), table.dtype),
        mesh=mesh,
    )
    def gather_kernel(tab_hbm, idx_hbm, out_hbm):
        def body(i_vmem, o_vmem):
            pltpu.sync_copy(tab_hbm.at[i_vmem.at[0]], o_vmem.at[0])

        pltpu.emit_pipeline(
            body,
            grid=(S, B // WINDOW),
            in_specs=[pl.BlockSpec((1, WINDOW), index_map=lambda s, b: (s, b))],
            out_specs=[
                pl.BlockSpec((1, WINDOW, 128), index_map=lambda s, b: (s, b, 0))
            ],
            core_axis_name=("core", "subcore"),
            dimension_semantics=(pltpu.PARALLEL, pltpu.PARALLEL),
        )(idx_hbm, out_hbm)

    rows3 = gather_kernel(tab_p, idx_t)
    out_sdb = _tail_transpose(rows3, B, S, D)
    return jnp.transpose(out_sdb, (2, 0, 1))


# speedup vs baseline: 2.3439x; 1.1152x over previous
import jax
import jax.numpy as jnp
from jax.experimental import pallas as pl
from jax.experimental.pallas import tpu as pltpu
from jax.experimental.pallas import tpu_sc as plsc

TCOLS = 16384
WINDOW = 128
WB = 4096


def _repack_table(tab_t):
    D, V = tab_t.shape

    def body(in_ref, out_ref):
        out_ref[:, :D] = in_ref[...].T

    return pl.pallas_call(
        body,
        grid=(pl.cdiv(V, TCOLS),),
        in_specs=[pl.BlockSpec((D, TCOLS), lambda i: (0, i))],
        out_specs=pl.BlockSpec((TCOLS, 128), lambda i: (i, 0)),
        out_shape=jax.ShapeDtypeStruct((V, 128), tab_t.dtype),
    )(tab_t)


def _tail_transpose(rows3, B, S, D):
    def body(in_ref, out_ref):
        x = in_ref[0]
        out_ref[0] = x[:, :D].T

    return pl.pallas_call(
        body,
        grid=(S, B // WB),
        in_specs=[pl.BlockSpec((1, WB, 128), lambda s, b: (s, b, 0))],
        out_specs=pl.BlockSpec((1, D, WB), lambda s, b: (s, 0, b)),
        out_shape=jax.ShapeDtypeStruct((S, D, B), rows3.dtype),
    )(rows3)


def kernel(input_tokens, table):
    B, S = input_tokens.shape
    V, D = table.shape

    tab_p = _repack_table(table.T)
    idx_t = input_tokens.T

    mesh = plsc.VectorSubcoreMesh(core_axis_name="core", subcore_axis_name="subcore")

    @pl.kernel(
        out_type=jax.ShapeDtypeStruct((S, B, 128), table.dtype),
        mesh=mesh,
    )
    def gather_kernel(tab_hbm, idx_hbm, out_hbm):
        def body(i_vmem, o_vmem):
            pltpu.sync_copy(tab_hbm.at[i_vmem.at[0]], o_vmem.at[0])

        pltpu.emit_pipeline(
            body,
            grid=(S, B // WINDOW),
            in_specs=[pl.BlockSpec((1, WINDOW), index_map=lambda s, b: (s, b))],
            out_specs=[
                pl.BlockSpec((1, WINDOW, 128), index_map=lambda s, b: (s, b, 0))
            ],
            core_axis_name=("core", "subcore"),
            dimension_semantics=(pltpu.PARALLEL, pltpu.PARALLEL),
        )(idx_hbm, out_hbm)

    rows3 = gather_kernel(tab_p, idx_t)
    out_sdb = _tail_transpose(rows3, B, S, D)
    return jnp.transpose(out_sdb, (2, 0, 1))


# speedup vs baseline: 2.4651x; 1.0517x over previous
import jax
import jax.numpy as jnp
from jax.experimental import pallas as pl
from jax.experimental.pallas import tpu as pltpu
from jax.experimental.pallas import tpu_sc as plsc

TCOLS = 20480
WINDOW = 128
WB = 8192


def _repack_table(tab_t):
    D, V = tab_t.shape

    def body(in_ref, out_ref):
        out_ref[:, :D] = in_ref[...].T

    return pl.pallas_call(
        body,
        grid=(pl.cdiv(V, TCOLS),),
        in_specs=[pl.BlockSpec((D, TCOLS), lambda i: (0, i))],
        out_specs=pl.BlockSpec((TCOLS, 128), lambda i: (i, 0)),
        out_shape=jax.ShapeDtypeStruct((V, 128), tab_t.dtype),
    )(tab_t)


def _tail_transpose(rows3, B, S, D):
    def body(in_ref, out_ref):
        x = in_ref[0]
        out_ref[0] = x[:, :D].T

    return pl.pallas_call(
        body,
        grid=(S, B // WB),
        in_specs=[pl.BlockSpec((1, WB, 128), lambda s, b: (s, b, 0))],
        out_specs=pl.BlockSpec((1, D, WB), lambda s, b: (s, 0, b)),
        out_shape=jax.ShapeDtypeStruct((S, D, B), rows3.dtype),
    )(rows3)


def kernel(input_tokens, table):
    B, S = input_tokens.shape
    V, D = table.shape

    tab_p = _repack_table(table.T)
    idx_t = input_tokens.T

    mesh = plsc.VectorSubcoreMesh(core_axis_name="core", subcore_axis_name="subcore")

    @pl.kernel(
        out_type=jax.ShapeDtypeStruct((S, B, 128), table.dtype),
        mesh=mesh,
    )
    def gather_kernel(tab_hbm, idx_hbm, out_hbm):
        def body(i_vmem, o_vmem):
            pltpu.sync_copy(tab_hbm.at[i_vmem.at[0]], o_vmem.at[0])

        pltpu.emit_pipeline(
            body,
            grid=(S, B // WINDOW),
            in_specs=[pl.BlockSpec((1, WINDOW), index_map=lambda s, b: (s, b))],
            out_specs=[
                pl.BlockSpec((1, WINDOW, 128), index_map=lambda s, b: (s, b, 0))
            ],
            core_axis_name=("core", "subcore"),
            dimension_semantics=(pltpu.PARALLEL, pltpu.PARALLEL),
        )(idx_hbm, out_hbm)

    rows3 = gather_kernel(tab_p, idx_t)
    out_sdb = _tail_transpose(rows3, B, S, D)
    return jnp.transpose(out_sdb, (2, 0, 1))


# speedup vs baseline: 2.5298x; 1.0263x over previous
import jax
import jax.numpy as jnp
from jax.experimental import pallas as pl
from jax.experimental.pallas import tpu as pltpu
from jax.experimental.pallas import tpu_sc as plsc

TCOLS = 20480
WINDOW = 128
WB = 8192


def _repack_table(tab_t):
    D, V = tab_t.shape

    def body(in_ref, out_ref):
        out_ref[:, :D] = in_ref[...].T

    return pl.pallas_call(
        body,
        grid=(pl.cdiv(V, TCOLS),),
        in_specs=[pl.BlockSpec((D, TCOLS), lambda i: (0, i))],
        out_specs=pl.BlockSpec((TCOLS, 128), lambda i: (i, 0)),
        out_shape=jax.ShapeDtypeStruct((V, 128), tab_t.dtype),
    )(tab_t)


def _tail_transpose(rows3, B, S, D):
    def body(in_ref, out_ref):
        x = in_ref[0]
        out_ref[0] = x[:, :D].T

    return pl.pallas_call(
        body,
        grid=(S, B // WB),
        in_specs=[pl.BlockSpec((1, WB, 128), lambda s, b: (s, b, 0))],
        out_specs=pl.BlockSpec((1, D, WB), lambda s, b: (s, 0, b)),
        out_shape=jax.ShapeDtypeStruct((S, D, B), rows3.dtype),
    )(rows3)


def kernel(input_tokens, table):
    B, S = input_tokens.shape
    V, D = table.shape

    tab_p = _repack_table(table.T)
    idx_t = input_tokens.T

    mesh = plsc.VectorSubcoreMesh(core_axis_name="core", subcore_axis_name="subcore")

    @pl.kernel(
        out_type=jax.ShapeDtypeStruct((S, B, 128), table.dtype),
        mesh=mesh,
    )
    def gather_kernel(tab_hbm, idx_hbm, out_hbm):
        def body(i_vmem, o_vmem):
            pltpu.sync_copy(
                tab_hbm.at[i_vmem.at[0, pl.ds(0, WINDOW)]],
                o_vmem.at[0, pl.ds(0, WINDOW)],
            )
            pltpu.sync_copy(
                tab_hbm.at[i_vmem.at[0, pl.ds(WINDOW, WINDOW)]],
                o_vmem.at[0, pl.ds(WINDOW, WINDOW)],
            )

        pltpu.emit_pipeline(
            body,
            grid=(S, B // (2 * WINDOW)),
            in_specs=[pl.BlockSpec((1, 2 * WINDOW), index_map=lambda s, b: (s, b))],
            out_specs=[
                pl.BlockSpec((1, 2 * WINDOW, 128), index_map=lambda s, b: (s, b, 0))
            ],
            core_axis_name=("core", "subcore"),
            dimension_semantics=(pltpu.PARALLEL, pltpu.PARALLEL),
        )(idx_hbm, out_hbm)

    rows3 = gather_kernel(tab_p, idx_t)
    out_sdb = _tail_transpose(rows3, B, S, D)
    return jnp.transpose(out_sdb, (2, 0, 1))
